# baseline (speedup 1.0000x reference)
import jax
import jax.numpy as jnp
from jax.experimental import pallas as pl
from jax.experimental.pallas import tpu as pltpu

_G, _K, _D, _B = 4, 8192, 32, 2048
_CC = 0.25
_BB = 256
_NB = _B // _BB


def _vq_kernel(x_ref, w_ref, oh_ref, emb_ref, qst_ref, loss_ref, perp_ref,
               counts_ref, w2_ref):
    b = pl.program_id(1)

    x = x_ref[0]
    w = w_ref[0]

    @pl.when(b == 0)
    def _():
        w2_ref[...] = jnp.sum(w * w, axis=1)[None, :]

    x2 = jnp.sum(x * x, axis=1, keepdims=True)
    mm = jax.lax.dot_general(x, w, (((1,), (1,)), ((), ())),
                             preferred_element_type=jnp.float32)
    dist = (x2 + w2_ref[...]) - 2.0 * mm

    mind = jnp.min(dist, axis=1, keepdims=True)
    iota = jax.lax.broadcasted_iota(jnp.int32, (_BB, _K), 1)
    idx = jnp.min(jnp.where(dist == mind, iota, jnp.int32(_K)), axis=1)
    oh = (iota == idx[:, None]).astype(jnp.float32)
    oh_ref[0] = oh

    q = jax.lax.dot_general(oh, w, (((1,), (0,)), ((), ())),
                            preferred_element_type=jnp.float32)
    emb_ref[0] = q
    qst_ref[0] = x + (q - x)

    first = jnp.logical_and(pl.program_id(0) == 0, b == 0)

    @pl.when(first)
    def _():
        loss_ref[...] = jnp.zeros((1, 1), jnp.float32)
        perp_ref[...] = jnp.zeros((1, 1), jnp.float32)

    d = q - x
    loss_ref[...] += jnp.sum(d * d).reshape(1, 1)

    colsum = jax.lax.dot_general(jnp.ones((1, _BB), jnp.float32), oh,
                                 (((1,), (0,)), ((), ())),
                                 preferred_element_type=jnp.float32)

    @pl.when(b == 0)
    def _():
        counts_ref[...] = colsum

    @pl.when(b != 0)
    def _():
        counts_ref[...] += colsum

    @pl.when(b == _NB - 1)
    def _():
        p = counts_ref[...] * (1.0 / _B)
        ent = jnp.sum(p * jnp.log(p + 1e-10))
        perp_ref[...] += jnp.exp(-ent).reshape(1, 1)


def kernel(inputs, W):
    xt = jnp.transpose(inputs, (1, 0, 2))

    grid = (_G, _NB)
    oh, emb, qst, loss_sum, perp_sum = pl.pallas_call(
        _vq_kernel,
        grid=grid,
        in_specs=[
            pl.BlockSpec((1, _BB, _D), lambda g, b: (g, b, 0)),
            pl.BlockSpec((1, _K, _D), lambda g, b: (g, 0, 0)),
        ],
        out_specs=[
            pl.BlockSpec((1, _BB, _K), lambda g, b: (g, b, 0)),
            pl.BlockSpec((1, _BB, _D), lambda g, b: (g, b, 0)),
            pl.BlockSpec((1, _BB, _D), lambda g, b: (g, b, 0)),
            pl.BlockSpec((1, 1), lambda g, b: (0, 0)),
            pl.BlockSpec((1, 1), lambda g, b: (0, 0)),
        ],
        out_shape=[
            jax.ShapeDtypeStruct((_G, _B, _K), jnp.float32),
            jax.ShapeDtypeStruct((_G, _B, _D), jnp.float32),
            jax.ShapeDtypeStruct((_G, _B, _D), jnp.float32),
            jax.ShapeDtypeStruct((1, 1), jnp.float32),
            jax.ShapeDtypeStruct((1, 1), jnp.float32),
        ],
        scratch_shapes=[pltpu.VMEM((1, _K), jnp.float32),
                        pltpu.VMEM((1, _K), jnp.float32)],
        compiler_params=pltpu.CompilerParams(
            dimension_semantics=("arbitrary", "arbitrary")),
    )(xt, W)

    avg_loss = (loss_sum[0, 0] * ((1.0 + _CC) / (_B * _D))) / _G
    avg_perplexity = perp_sum[0, 0] / _G
    quantized_all = jnp.transpose(qst, (1, 0, 2))
    return (avg_loss, quantized_all, avg_perplexity, emb, oh)


# speedup vs baseline: 1.1619x; 1.1619x over previous
import jax
import jax.numpy as jnp
from jax.experimental import pallas as pl
from jax.experimental.pallas import tpu as pltpu

_G, _K, _D, _B = 4, 8192, 32, 2048
_CC = 0.25
_BB = 128
_NB = _B // _BB


def _vq_kernel(x_ref, w_ref, oh_ref, emb_ref, qst_ref, loss_ref, perp_ref,
               counts_ref, w2_ref):
    b = pl.program_id(1)

    x = x_ref[0]
    w = w_ref[0]

    @pl.when(b == 0)
    def _():
        w2_ref[...] = jnp.sum(w * w, axis=1)[None, :]

    x2 = jnp.sum(x * x, axis=1, keepdims=True)
    mm = jax.lax.dot_general(x, w, (((1,), (1,)), ((), ())),
                             preferred_element_type=jnp.float32)
    dist = (x2 + w2_ref[...]) - 2.0 * mm

    mind = jnp.min(dist, axis=1, keepdims=True)
    iota = jax.lax.broadcasted_iota(jnp.int32, (_BB, _K), 1)
    idx = jnp.min(jnp.where(dist == mind, iota, jnp.int32(_K)), axis=1)
    oh = (iota == idx[:, None]).astype(jnp.float32)
    oh_ref[0] = oh

    q = jax.lax.dot_general(oh, w, (((1,), (0,)), ((), ())),
                            preferred_element_type=jnp.float32)
    emb_ref[0] = q
    qst_ref[0] = x + (q - x)

    first = jnp.logical_and(pl.program_id(0) == 0, b == 0)

    @pl.when(first)
    def _():
        loss_ref[...] = jnp.zeros((1, 1), jnp.float32)
        perp_ref[...] = jnp.zeros((1, 1), jnp.float32)

    d = q - x
    loss_ref[...] += jnp.sum(d * d).reshape(1, 1)

    colsum = jax.lax.dot_general(jnp.ones((1, _BB), jnp.float32), oh,
                                 (((1,), (0,)), ((), ())),
                                 preferred_element_type=jnp.float32)

    @pl.when(b == 0)
    def _():
        counts_ref[...] = colsum

    @pl.when(b != 0)
    def _():
        counts_ref[...] += colsum

    @pl.when(b == _NB - 1)
    def _():
        p = counts_ref[...] * (1.0 / _B)
        ent = jnp.sum(p * jnp.log(p + 1e-10))
        perp_ref[...] += jnp.exp(-ent).reshape(1, 1)


def kernel(inputs, W):
    xt = jnp.transpose(inputs, (1, 0, 2))

    grid = (_G, _NB)
    oh, emb, qst, loss_sum, perp_sum = pl.pallas_call(
        _vq_kernel,
        grid=grid,
        in_specs=[
            pl.BlockSpec((1, _BB, _D), lambda g, b: (g, b, 0)),
            pl.BlockSpec((1, _K, _D), lambda g, b: (g, 0, 0)),
        ],
        out_specs=[
            pl.BlockSpec((1, _BB, _K), lambda g, b: (g, b, 0)),
            pl.BlockSpec((1, _BB, _D), lambda g, b: (g, b, 0)),
            pl.BlockSpec((1, _BB, _D), lambda g, b: (g, b, 0)),
            pl.BlockSpec((1, 1), lambda g, b: (0, 0)),
            pl.BlockSpec((1, 1), lambda g, b: (0, 0)),
        ],
        out_shape=[
            jax.ShapeDtypeStruct((_G, _B, _K), jnp.float32),
            jax.ShapeDtypeStruct((_G, _B, _D), jnp.float32),
            jax.ShapeDtypeStruct((_G, _B, _D), jnp.float32),
            jax.ShapeDtypeStruct((1, 1), jnp.float32),
            jax.ShapeDtypeStruct((1, 1), jnp.float32),
        ],
        scratch_shapes=[pltpu.VMEM((1, _K), jnp.float32),
                        pltpu.VMEM((1, _K), jnp.float32)],
        compiler_params=pltpu.CompilerParams(
            dimension_semantics=("arbitrary", "arbitrary")),
    )(xt, W)

    avg_loss = (loss_sum[0, 0] * ((1.0 + _CC) / (_B * _D))) / _G
    avg_perplexity = perp_sum[0, 0] / _G
    quantized_all = jnp.transpose(qst, (1, 0, 2))
    return (avg_loss, quantized_all, avg_perplexity, emb, oh)
